# baseline (speedup 1.0000x reference)
import functools

import jax
import jax.numpy as jnp
from jax import lax
from jax.experimental import pallas as pl
from jax.experimental.pallas import tpu as pltpu
from jax.experimental.pallas import tpu_sc as plsc

N_NODES = 370
HID = 64
N_LAYERS = 8
T_STEPS = 32
B_SIZE = 4
N_EDGES = 11840
K_TOP = 37
NPAD = 512
TPAD = 384
ROWSP = T_STEPS * TPAD

EDGES_PAD = 12288
EPT = EDGES_PAD // 32
WCH = (N_NODES * NPAD) // 16


def _sc_build_wdense(srcp, dstp, wp):
    mesh = plsc.VectorSubcoreMesh(core_axis_name="c", subcore_axis_name="s")

    @functools.partial(
        pl.kernel,
        mesh=mesh,
        out_type=jax.ShapeDtypeStruct((2 * N_NODES * NPAD,), jnp.float32),
        scratch_types=[
            pltpu.VMEM((EPT,), jnp.int32),
            pltpu.VMEM((EPT,), jnp.int32),
            pltpu.VMEM((EPT,), jnp.float32),
            pltpu.VMEM((3, 128), jnp.int32),
            pltpu.VMEM((3, 128), jnp.float32),
            pltpu.VMEM((WCH,), jnp.float32),
            pltpu.VMEM_SHARED((N_NODES * NPAD,), jnp.float32),
        ],
    )
    def k(src_hbm, dst_hbm, w_hbm, out_hbm, sv, dv, wv, idx2, val2, zv, acc):
        c = lax.axis_index("c")
        s = lax.axis_index("s")
        gid = c * 16 + s

        def zbody(j, carry):
            zv[pl.ds(j * 16, 16)] = jnp.zeros((16,), jnp.float32)
            return carry

        lax.fori_loop(0, WCH // 16, zbody, 0)
        pltpu.sync_copy(zv, acc.at[pl.ds(s * WCH, WCH)])

        base = gid * EPT
        pltpu.sync_copy(src_hbm.at[pl.ds(base, EPT)], sv)
        pltpu.sync_copy(dst_hbm.at[pl.ds(base, EPT)], dv)
        pltpu.sync_copy(w_hbm.at[pl.ds(base, EPT)], wv)
        for r in range(3):
            for q in range(8):
                o = r * 128 + q * 16
                idx2[r, pl.ds(q * 16, 16)] = (
                    sv[pl.ds(o, 16)] * NPAD + dv[pl.ds(o, 16)]
                )
                val2[r, pl.ds(q * 16, 16)] = wv[pl.ds(o, 16)]
        plsc.subcore_barrier()
        for r in range(3):
            pltpu.sync_copy(val2.at[r], acc.at[idx2.at[r]], add=True)
        plsc.subcore_barrier()
        pltpu.sync_copy(acc.at[pl.ds(s * WCH, WCH)], zv)
        pltpu.sync_copy(
            zv, out_hbm.at[pl.ds(c * (N_NODES * NPAD) + s * WCH, WCH)]
        )

    return k(srcp, dstp, wp)


def _adj_kernel(wd2_ref, es_ref, et_ref, out_ref):
    f32 = jnp.float32
    n = N_NODES
    iota_r = lax.broadcasted_iota(jnp.int32, (n, n), 0)
    iota_c = lax.broadcasted_iota(jnp.int32, (n, n), 1)
    eye = (iota_r == iota_c).astype(f32)

    def t2r(m):
        return lax.dot_general(
            eye, m, (((1,), (1,)), ((), ())), preferred_element_type=f32
        )

    wd = wd2_ref[0] + wd2_ref[1]
    wsd = wd[:, :n]
    deg_r = jnp.sum(wsd, axis=0, keepdims=True) + 1.0
    dinv_r = lax.rsqrt(deg_r)
    dinv_c = t2r(dinv_r)
    a_fixed_t = dinv_c * wsd * dinv_r + eye * (dinv_c * dinv_c)

    for l in range(N_LAYERS):
        s = lax.dot_general(
            es_ref[l], et_ref[l], (((1,), (1,)), ((), ())),
            preferred_element_type=f32,
        )
        v0 = jnp.maximum(s, 0.0)

        def body(j, carry):
            v, maskf = carry
            m = jnp.max(v, axis=1, keepdims=True)
            cand = jnp.where(v >= m, iota_c, jnp.int32(10**9))
            sel = jnp.min(cand, axis=1, keepdims=True)
            hit = iota_c == sel
            maskf = maskf + jnp.where(hit, 1.0, 0.0)
            v = jnp.where(hit, -1e30, v)
            return v, maskf

        _, maskf = lax.fori_loop(
            0, K_TOP, body, (v0, jnp.zeros((n, n), f32))
        )
        mx = jnp.max(v0, axis=1, keepdims=True)
        p = jnp.exp(v0 - mx)
        z = jnp.sum(p, axis=1, keepdims=True)
        wa = jnp.where(maskf > 0.0, p / z, 0.0)
        dega_r = jnp.sum(wa, axis=0, keepdims=True) + 1.0
        dinva_r = lax.rsqrt(dega_r)
        dinva_c = t2r(dinva_r)
        a_adapt_t = dinva_c * wa * dinva_r + eye * (dinva_c * dinva_c)
        out_ref[l, 0:N_NODES, :] = a_fixed_t
        out_ref[l, N_NODES:TPAD, :] = jnp.zeros((TPAD - N_NODES, n), f32)
        out_ref[l, TPAD : TPAD + N_NODES, :] = a_adapt_t
        out_ref[l, TPAD + N_NODES : 2 * TPAD, :] = jnp.zeros(
            (TPAD - N_NODES, n), f32
        )


def _layers_kernel(
    x_ref, aa_ref, wfa_ref, bsum_ref, w01_ref, bconv_ref,
    wrs_ref, brs_ref, lng_ref, lnb_ref, inpw_ref, inpb_ref,
    e1w_ref, e1b_ref, e2w_ref, e2b_ref, out_ref, big_s, pi_s,
):
    f32 = jnp.float32

    def mm(a, b):
        return jnp.dot(a, b, preferred_element_type=f32)

    def gelu(v):
        return v * 0.5 * (1.0 + lax.erf(v * 0.7071067811865476))

    x = mm(inpw_ref[:], x_ref[0]) + inpb_ref[:]
    out_ref[0] = jnp.zeros((HID, ROWSP), f32)

    for i in range(N_LAYERS):
        d = 2 ** (i % 4)
        big_s[:, :] = mm(wfa_ref[i], x)
        aai = aa_ref[i]

        def tbody(t, carry):
            pi_s[:, pl.ds(2 * t * TPAD, TPAD)] = big_s[
                0:HID, pl.ds(t * TPAD, TPAD)
            ]
            pi_s[:, pl.ds((2 * t + 1) * TPAD, TPAD)] = big_s[
                HID : 2 * HID, pl.ds(t * TPAD, TPAD)
            ]
            big_s[HID : 2 * HID, pl.ds(t * TPAD, N_NODES)] = mm(
                pi_s[:, pl.ds(2 * t * TPAD, 2 * TPAD)], aai
            )
            return carry

        lax.fori_loop(0, T_STEPS, tbody, 0)
        hn = big_s[HID : 2 * HID, :] + bsum_ref[i]
        sh = jnp.concatenate(
            [jnp.zeros((HID, d * TPAD), f32), hn[:, : ROWSP - d * TPAD]],
            axis=1,
        )
        big_s[0:HID, :] = sh
        big_s[HID : 2 * HID, :] = hn
        fg = mm(w01_ref[i], big_s[:, :]) + bconv_ref[i]
        hg = jnp.tanh(fg[0:HID, :]) * jax.nn.sigmoid(fg[HID : 2 * HID, :])
        rs = mm(wrs_ref[i], hg) + brs_ref[i]
        out_ref[0] = out_ref[0] + rs[HID : 2 * HID, :]
        r = rs[0:HID, :] + x
        mu = jnp.mean(r, axis=0, keepdims=True)
        dev = r - mu
        var = jnp.mean(dev * dev, axis=0, keepdims=True)
        x = dev * lax.rsqrt(var + 1e-5) * lng_ref[i] + lnb_ref[i]

    h = gelu(out_ref[0])
    h = gelu(mm(e1w_ref[:], h) + e1b_ref[:])
    h = mm(e2w_ref[:], h) + e2b_ref[:]
    out_ref[0] = h


def _head_kernel(xt_ref, w1_ref, b1_ref, w2_ref, b2_ref, out_ref):
    f32 = jnp.float32
    h = jnp.dot(w1_ref[:], xt_ref[:], preferred_element_type=f32) + b1_ref[:]
    h = h * 0.5 * (1.0 + lax.erf(h * 0.7071067811865476))
    out_ref[:, :] = (
        jnp.dot(w2_ref[:], h, preferred_element_type=f32) + b2_ref[:]
    )


def kernel(x, edge_index, edge_weight, params):
    f32 = jnp.float32
    pad = EDGES_PAD - N_EDGES
    srcp = jnp.concatenate([edge_index[0], jnp.zeros((pad,), jnp.int32)])
    dstp = jnp.concatenate([edge_index[1], jnp.zeros((pad,), jnp.int32)])
    wp = jnp.concatenate([edge_weight.astype(f32), jnp.zeros((pad,), f32)])

    wd2 = _sc_build_wdense(srcp, dstp, wp).reshape(2, N_NODES, NPAD)

    lps = params["layers"]
    es = jnp.stack([lp["emb_src"] for lp in lps])
    et = jnp.stack([lp["emb_tgt"] for lp in lps])

    aa = pl.pallas_call(
        _adj_kernel,
        out_shape=jax.ShapeDtypeStruct((N_LAYERS, 2 * TPAD, N_NODES), f32),
    )(wd2, es, et)

    wfa = jnp.stack(
        [
            jnp.concatenate([lp["gcn_fixed_W"], lp["gcn_adapt_W"]], axis=0)
            for lp in lps
        ]
    )
    bsum = jnp.stack(
        [lp["gcn_fixed_b"] + lp["gcn_adapt_b"] for lp in lps]
    ).reshape(N_LAYERS, HID, 1)
    w01 = jnp.stack(
        [
            jnp.concatenate(
                [
                    jnp.concatenate(
                        [lp["filter_w"][:, :, 0], lp["gate_w"][:, :, 0]],
                        axis=0,
                    ),
                    jnp.concatenate(
                        [lp["filter_w"][:, :, 1], lp["gate_w"][:, :, 1]],
                        axis=0,
                    ),
                ],
                axis=1,
            )
            for lp in lps
        ]
    )
    bconv = jnp.stack(
        [jnp.concatenate([lp["filter_b"], lp["gate_b"]]) for lp in lps]
    ).reshape(N_LAYERS, 2 * HID, 1)
    wrs = jnp.stack(
        [
            jnp.concatenate(
                [lp["res_w"][:, :, 0], lp["skip_w"][:, :, 0]], axis=0
            )
            for lp in lps
        ]
    )
    brs = jnp.stack(
        [jnp.concatenate([lp["res_b"], lp["skip_b"]]) for lp in lps]
    ).reshape(N_LAYERS, 2 * HID, 1)
    lng = jnp.stack([lp["ln_g"] for lp in lps]).reshape(N_LAYERS, HID, 1)
    lnb = jnp.stack([lp["ln_b"] for lp in lps]).reshape(N_LAYERS, HID, 1)

    xt = jnp.pad(
        jnp.transpose(x, (0, 3, 1, 2)), ((0, 0), (0, 0), (0, 0), (0, TPAD - N_NODES))
    ).reshape(B_SIZE, HID, ROWSP)

    full = lambda shp: pl.BlockSpec(shp, lambda b: tuple(0 for _ in shp))
    skipact = pl.pallas_call(
        _layers_kernel,
        grid=(B_SIZE,),
        in_specs=[
            pl.BlockSpec((1, HID, ROWSP), lambda b: (b, 0, 0)),
            full((N_LAYERS, 2 * TPAD, N_NODES)),
            full((N_LAYERS, 2 * HID, HID)),
            full((N_LAYERS, HID, 1)),
            full((N_LAYERS, 2 * HID, 2 * HID)),
            full((N_LAYERS, 2 * HID, 1)),
            full((N_LAYERS, 2 * HID, HID)),
            full((N_LAYERS, 2 * HID, 1)),
            full((N_LAYERS, HID, 1)),
            full((N_LAYERS, HID, 1)),
            full((HID, HID)),
            full((HID, 1)),
            full((HID, HID)),
            full((HID, 1)),
            full((HID, HID)),
            full((HID, 1)),
        ],
        out_specs=pl.BlockSpec((1, HID, ROWSP), lambda b: (b, 0, 0)),
        out_shape=jax.ShapeDtypeStruct((B_SIZE, HID, ROWSP), f32),
        scratch_shapes=[
            pltpu.VMEM((2 * HID, ROWSP), f32),
            pltpu.VMEM((HID, 2 * ROWSP), f32),
        ],
    )(
        xt, aa, wfa, bsum, w01, bconv, wrs, brs, lng, lnb,
        params["inp_W"], params["inp_b"].reshape(HID, 1),
        params["end1_W"], params["end1_b"].reshape(HID, 1),
        params["end2_W"], params["end2_b"].reshape(HID, 1),
    )

    hf = (
        skipact.reshape(B_SIZE, HID, T_STEPS, TPAD)[:, :, :, :N_NODES]
        .transpose(3, 1, 0, 2)
        .reshape(N_NODES * HID, B_SIZE * T_STEPS)
    )
    outt = pl.pallas_call(
        _head_kernel,
        out_shape=jax.ShapeDtypeStruct((HID, B_SIZE * T_STEPS), f32),
    )(
        hf,
        params["head1_W"],
        params["head1_b"].reshape(256, 1),
        params["head2_W"],
        params["head2_b"].reshape(HID, 1),
    )
    return outt.T.reshape(B_SIZE, T_STEPS, HID)
